# baseline (speedup 1.0000x reference)
import functools

import jax
import jax.numpy as jnp
from jax.experimental import pallas as pl
from jax.experimental.pallas import tpu as pltpu

N_IMG = 16
L = 16384
N = N_IMG * L
K = max(1, int(L * 0.02))
N_POS = N_IMG * K
C_CLS = 17
C_BOX = 5
R128 = N // 128
BR = 64
NB = R128 // BR


def _sigmoid(x):
    return jax.nn.sigmoid(x)


def _bce(p, t):
    lp = jnp.clip(jnp.log(p), -100.0, None)
    l1p = jnp.clip(jnp.log(1.0 - p), -100.0, None)
    return -(t * lp + (1.0 - t) * l1p)


def _loss_kernel(t_cls_ref, s_cls_ref, t3_ref, t_box_ref, s_box_ref,
                 t_cent_ref, s_cent_ref,
                 out_cls_ref, out_box_ref, out_cent_ref,
                 w17_ref, w5_ref, max_ref, diff_ref, c_ref, d_ref, acc_ref):
    i = pl.program_id(0)

    @pl.when(i == 0)
    def _init():
        r17 = jax.lax.broadcasted_iota(jnp.int32, (128 * C_CLS, 128), 0)
        c17 = jax.lax.broadcasted_iota(jnp.int32, (128 * C_CLS, 128), 1)
        w17_ref[...] = ((r17 >= C_CLS * c17) & (r17 < C_CLS * c17 + C_CLS)
                        ).astype(jnp.float32)
        r5 = jax.lax.broadcasted_iota(jnp.int32, (128 * C_BOX, 128), 0)
        c5 = jax.lax.broadcasted_iota(jnp.int32, (128 * C_BOX, 128), 1)
        w5_ref[...] = ((r5 >= C_BOX * c5) & (r5 < C_BOX * c5 + C_BOX)
                       ).astype(jnp.float32)
        acc_ref[0, 0] = 0.0

    rows = pl.ds(i * BR, BR)

    t = t_cls_ref[...]
    s = s_cls_ref[...]
    s_sig = _sigmoid(s)
    t_sig = _sigmoid(t)
    ls = jnp.clip(jnp.log(s_sig), -100.0, None)
    l1s = jnp.clip(jnp.log(1.0 - s_sig), -100.0, None)
    neg = -l1s * (s_sig * s_sig)
    dts = t_sig - s_sig
    pos = -(t_sig * ls + (1.0 - t_sig) * l1s) * (dts * dts)
    acc_ref[0, 0] += jnp.sum(neg)
    diff_ref[rows, :] = jnp.dot(pos - neg, w17_ref[...],
                                preferred_element_type=jnp.float32)

    max_ref[rows, :] = _sigmoid(jnp.max(t3_ref[...], axis=2))

    dbox = jnp.abs(s_box_ref[...] - t_box_ref[...])
    sl1 = jnp.where(dbox < 1.0, 0.5 * dbox * dbox, dbox - 0.5)
    tcs = _sigmoid(t_cent_ref[...])
    c_ref[rows, :] = jnp.dot(sl1, w5_ref[...],
                             preferred_element_type=jnp.float32) * tcs

    d_ref[rows, :] = _bce(_sigmoid(s_cent_ref[...]), tcs)

    @pl.when(i == NB - 1)
    def _finalize():
        m3 = max_ref[...].reshape(N_IMG, R128 // N_IMG, 128)
        bits = jax.lax.bitcast_convert_type(m3, jnp.int32)

        def body(_, lohi):
            lo, hi = lohi
            mid = lo + (hi - lo) // 2
            cnt = jnp.sum((bits >= mid[:, None, None]).astype(jnp.int32),
                          axis=(1, 2))
            ge = cnt >= K
            return (jnp.where(ge, mid, lo), jnp.where(ge, hi, mid))

        lo0 = jnp.zeros((N_IMG,), jnp.int32)
        hi0 = jnp.full((N_IMG,), 0x3F800001, jnp.int32)
        lo, _ = jax.lax.fori_loop(0, 31, body, (lo0, hi0))
        thr = jax.lax.bitcast_convert_type(lo, jnp.float32)[:, None, None]

        gt = m3 > thr
        sum_gt = jnp.sum(jnp.where(gt, m3, 0.0), axis=(1, 2))
        cnt_gt = jnp.sum(gt.astype(jnp.float32), axis=(1, 2))
        sum_top = jnp.sum(sum_gt + (K - cnt_gt) * thr[:, 0, 0])
        fg_num = 1e-06 + sum_top

        mask = m3 >= thr
        diff3 = diff_ref[...].reshape(N_IMG, R128 // N_IMG, 128)
        c3 = c_ref[...].reshape(N_IMG, R128 // N_IMG, 128)
        d3 = d_ref[...].reshape(N_IMG, R128 // N_IMG, 128)
        pos_diff = jnp.sum(jnp.where(mask, diff3, 0.0))
        bbox_sum = jnp.sum(jnp.where(mask, c3, 0.0))
        cent_sum = jnp.sum(jnp.where(mask, d3, 0.0))

        out_cls_ref[0, 0] = (acc_ref[0, 0] + pos_diff) / fg_num
        out_box_ref[0, 0] = bbox_sum / (N_POS * C_BOX)
        out_cent_ref[0, 0] = cent_sum / N_POS


@jax.jit
def _run(t_cls, t_box, t_cent, s_cls, s_box, s_cent):
    t2 = t_cls.reshape(R128, 128 * C_CLS)
    s2 = s_cls.reshape(R128, 128 * C_CLS)
    t3 = t_cls.reshape(R128, 128, C_CLS)
    tb = t_box.reshape(R128, 128 * C_BOX)
    sb = s_box.reshape(R128, 128 * C_BOX)
    tc = t_cent.reshape(R128, 128)
    sc = s_cent.reshape(R128, 128)

    out = pl.pallas_call(
        _loss_kernel,
        grid=(NB,),
        in_specs=[
            pl.BlockSpec((BR, 128 * C_CLS), lambda i: (i, 0)),
            pl.BlockSpec((BR, 128 * C_CLS), lambda i: (i, 0)),
            pl.BlockSpec((BR, 128, C_CLS), lambda i: (i, 0, 0)),
            pl.BlockSpec((BR, 128 * C_BOX), lambda i: (i, 0)),
            pl.BlockSpec((BR, 128 * C_BOX), lambda i: (i, 0)),
            pl.BlockSpec((BR, 128), lambda i: (i, 0)),
            pl.BlockSpec((BR, 128), lambda i: (i, 0)),
        ],
        out_specs=[
            pl.BlockSpec(memory_space=pltpu.SMEM),
            pl.BlockSpec(memory_space=pltpu.SMEM),
            pl.BlockSpec(memory_space=pltpu.SMEM),
        ],
        out_shape=[jax.ShapeDtypeStruct((1, 1), jnp.float32)] * 3,
        scratch_shapes=[
            pltpu.VMEM((128 * C_CLS, 128), jnp.float32),
            pltpu.VMEM((128 * C_BOX, 128), jnp.float32),
            pltpu.VMEM((R128, 128), jnp.float32),
            pltpu.VMEM((R128, 128), jnp.float32),
            pltpu.VMEM((R128, 128), jnp.float32),
            pltpu.VMEM((R128, 128), jnp.float32),
            pltpu.SMEM((1, 1), jnp.float32),
        ],
    )(t2, s2, t3, tb, sb, tc, sc)
    return out[0][0, 0], out[1][0, 0], out[2][0, 0]


def kernel(t_cls_scores, t_bbox_preds, t_centernesses, s_cls_scores,
           s_bbox_preds, s_centernesses, num_per_img):
    return _run(t_cls_scores, t_bbox_preds, t_centernesses,
                s_cls_scores, s_bbox_preds, s_centernesses)
